# baseline (speedup 1.0000x reference)
import functools

import jax
import jax.numpy as jnp
from jax import lax
from jax.experimental import pallas as pl
from jax.experimental.pallas import tpu as pltpu
from jax.experimental.pallas import tpu_sc as plsc

_K = 64
_NCAND = 32768
_NSAMP = 1024
_LANES = 16
_ROWS_PER_SAMPLE = _NCAND // _LANES


def _topk_body(x_ref, vals_ref, idx_ref, scratch_ref):
    x = x_ref[...]
    scratch_ref[...] = x
    rm0 = jnp.max(x.reshape(8, 32, 128), axis=2)
    rowiota = (
        lax.broadcasted_iota(jnp.int32, (8, 32), 0) * 32
        + lax.broadcasted_iota(jnp.int32, (8, 32), 1)
    )
    coliota = lax.broadcasted_iota(jnp.int32, (1, 128), 1)
    big = jnp.int32(0x7FFFFFFF)

    def step(i, rm):
        m = jnp.max(rm)
        r = jnp.min(jnp.where(rm == m, rowiota, big))
        row = scratch_ref[pl.ds(r, 1), :]
        c = jnp.min(jnp.where(row == m, coliota, big))
        vals_ref[i] = m
        idx_ref[i] = r * 128 + c
        newrow = jnp.where(coliota == c, -jnp.inf, row)
        scratch_ref[pl.ds(r, 1), :] = newrow
        return jnp.where(rowiota == r, jnp.max(newrow), rm)

    lax.fori_loop(0, _K, step, rm0)


def _topk(coef):
    x = coef.reshape(256, 128)
    return pl.pallas_call(
        _topk_body,
        out_shape=(
            jax.ShapeDtypeStruct((_K,), jnp.float32),
            jax.ShapeDtypeStruct((_K,), jnp.int32),
        ),
        in_specs=[pl.BlockSpec(memory_space=pltpu.VMEM)],
        out_specs=(
            pl.BlockSpec(memory_space=pltpu.SMEM),
            pl.BlockSpec(memory_space=pltpu.SMEM),
        ),
        scratch_shapes=[pltpu.VMEM((256, 128), jnp.float32)],
    )(x)


def _sc_gather_call(table_flat, idx, vals):
    info = plsc.get_sparse_core_info()
    nc, ns = info.num_cores, info.num_subcores
    nw = nc * ns
    spw = _NSAMP // nw
    n_el = spw * _K
    chunk = 128
    n_chunks = n_el // chunk

    mesh = plsc.VectorSubcoreMesh(core_axis_name="c", subcore_axis_name="s")

    @functools.partial(
        pl.kernel,
        mesh=mesh,
        out_type=jax.ShapeDtypeStruct((_NSAMP, _LANES), jnp.float32),
        scratch_types=[
            pltpu.VMEM((_K,), jnp.int32),
            pltpu.VMEM((_K,), jnp.float32),
            pltpu.VMEM((n_el,), jnp.int32),
            pltpu.VMEM((n_el,), jnp.float32),
            pltpu.VMEM((spw, _LANES), jnp.float32),
            pltpu.SemaphoreType.DMA,
        ],
    )
    def k(table_hbm, idx_hbm, val_hbm, out_hbm, idx_v, val_v, flat_v,
          gath_v, out_v, sem):
        wid = lax.axis_index("s") * nc + lax.axis_index("c")
        base_sample = wid * spw

        pltpu.sync_copy(idx_hbm, idx_v)
        pltpu.sync_copy(val_hbm, val_v)

        idx_chunks = []
        for kc in range(_K // _LANES):
            iv = idx_v[pl.ds(kc * _LANES, _LANES)]
            idx_chunks.append(
                lax.shift_left(lax.shift_right_logical(iv, 7), 10)
                + jnp.bitwise_and(iv, 127)
            )

        def build(s, _):
            sg = base_sample + s
            sbase = (
                lax.shift_left(lax.shift_right_logical(sg, 3), 18)
                + lax.shift_left(jnp.bitwise_and(sg, 7), 7)
            )
            for kc in range(_K // _LANES):
                flat_v[pl.ds(s * _K + kc * _LANES, _LANES)] = (
                    sbase + idx_chunks[kc]
                )
            return 0

        lax.fori_loop(0, spw, build, 0)

        copies = []
        for c in range(n_chunks):
            copies.append(
                pltpu.async_copy(
                    table_hbm.at[flat_v.at[pl.ds(c * chunk, chunk)]],
                    gath_v.at[pl.ds(c * chunk, chunk)],
                    sem,
                )
            )
        for cp in copies:
            cp.wait()

        pvals = [
            val_v[pl.ds(kc * _LANES, _LANES)] for kc in range(_K // _LANES)
        ]

        def extract(s, _):
            acc = jnp.zeros((_LANES,), jnp.float32)
            for kc in range(_K // _LANES):
                v = gath_v[pl.ds(s * _K + kc * _LANES, _LANES)]
                acc = acc + v * pvals[kc]
            out_v[s, :] = acc
            return 0

        lax.fori_loop(0, spw, extract, 0)

        pltpu.sync_copy(out_v, out_hbm.at[pl.ds(base_sample, spw)])

    return k(table_flat, idx, vals)


def _finish_body(p_ref, out_ref):
    p = p_ref[...]
    s = jnp.sum(p, axis=1, keepdims=True)
    out_ref[0, 0] = jnp.sum(jnp.log(s)) * (1.0 / _NSAMP)


def _finish(partials):
    out = pl.pallas_call(
        _finish_body,
        out_shape=jax.ShapeDtypeStruct((1, 1), jnp.float32),
        in_specs=[pl.BlockSpec(memory_space=pltpu.VMEM)],
        out_specs=pl.BlockSpec(memory_space=pltpu.SMEM),
    )(partials)
    return out.reshape(())


def kernel(inputs, coef_output_avg, sample_prob_all):
    del inputs
    vals, idx = _topk(coef_output_avg)
    table_flat = (
        sample_prob_all.reshape(_NSAMP // 8, 8, _NCAND // 128, 128)
        .transpose(0, 2, 1, 3)
        .reshape(_NSAMP * _NCAND)
    )
    partials = _sc_gather_call(table_flat, idx, vals)
    return _finish(partials)


# speedup vs baseline: 1.4988x; 1.4988x over previous
import functools

import jax
import jax.numpy as jnp
from jax import lax
from jax.experimental import pallas as pl
from jax.experimental.pallas import tpu as pltpu
from jax.experimental.pallas import tpu_sc as plsc

_K = 64
_NCAND = 32768
_NSAMP = 1024
_LANES = 16
_ROWS_PER_SAMPLE = _NCAND // _LANES


_NSLAB = 4


def _topk_body(x_ref, vals_ref, idx_ref):
    x3 = x_ref[...].reshape(_NSLAB, 64, 128)
    rows = lax.broadcasted_iota(jnp.int32, (_NSLAB, 64, 128), 1)
    slabs = lax.broadcasted_iota(jnp.int32, (_NSLAB, 64, 128), 0)
    cols = lax.broadcasted_iota(jnp.int32, (_NSLAB, 64, 128), 2)
    flat3 = (slabs * 64 + rows) * 128 + cols
    col64 = lax.broadcasted_iota(jnp.int32, (_NSLAB, _K), 1)
    big = jnp.int32(0x7FFFFFFF)

    def step(i, carry):
        x, cv, ci = carry
        m = jnp.max(x, axis=(1, 2), keepdims=True)
        j = jnp.min(
            jnp.where(x == m, flat3, big), axis=(1, 2), keepdims=True
        )
        cv = jnp.where(col64 == i, m[:, :, 0], cv)
        ci = jnp.where(col64 == i, j[:, :, 0], ci)
        x = jnp.where(flat3 == j, -jnp.inf, x)
        return x, cv, ci

    _, cv, ci = lax.fori_loop(
        0, _K, step,
        (
            x3,
            jnp.zeros((_NSLAB, _K), jnp.float32),
            jnp.zeros((_NSLAB, _K), jnp.int32),
        ),
    )

    v2 = jnp.concatenate(
        [cv[s : s + 1, :] for s in range(_NSLAB)], axis=1
    )
    i2 = jnp.concatenate(
        [ci[s : s + 1, :] for s in range(_NSLAB)], axis=1
    )
    v1 = jnp.transpose(v2, (1, 0))
    i1 = jnp.transpose(i2, (1, 0))
    beats = jnp.logical_or(
        v2 > v1, jnp.logical_and(v2 == v1, i2 < i1)
    )
    rank = jnp.sum(jnp.where(beats, 1, 0), axis=1, keepdims=True)
    k64 = lax.broadcasted_iota(jnp.int32, (1, _K), 1)
    onehot = rank == k64
    vals_ref[...] = jnp.sum(
        jnp.where(onehot, v1, 0.0), axis=0, keepdims=True
    )
    idx_ref[...] = jnp.sum(
        jnp.where(onehot, i1, 0), axis=0, keepdims=True
    )


def _topk(coef):
    x = coef.reshape(256, 128)
    vals, idx = pl.pallas_call(
        _topk_body,
        out_shape=(
            jax.ShapeDtypeStruct((1, _K), jnp.float32),
            jax.ShapeDtypeStruct((1, _K), jnp.int32),
        ),
        in_specs=[pl.BlockSpec(memory_space=pltpu.VMEM)],
        out_specs=(
            pl.BlockSpec(memory_space=pltpu.VMEM),
            pl.BlockSpec(memory_space=pltpu.VMEM),
        ),
    )(x)
    return vals.reshape(_K), idx.reshape(_K)


def _sc_gather_call(table_flat, idx, vals):
    info = plsc.get_sparse_core_info()
    nc, ns = info.num_cores, info.num_subcores
    nw = nc * ns
    spw = _NSAMP // nw
    n_el = spw * _K
    chunk = 128
    n_chunks = n_el // chunk

    mesh = plsc.VectorSubcoreMesh(core_axis_name="c", subcore_axis_name="s")

    @functools.partial(
        pl.kernel,
        mesh=mesh,
        out_type=jax.ShapeDtypeStruct((_NSAMP, _LANES), jnp.float32),
        scratch_types=[
            pltpu.VMEM((_K,), jnp.int32),
            pltpu.VMEM((_K,), jnp.float32),
            pltpu.VMEM((n_el,), jnp.int32),
            pltpu.VMEM((n_el,), jnp.float32),
            pltpu.VMEM((spw, _LANES), jnp.float32),
            pltpu.SemaphoreType.DMA,
        ],
    )
    def k(table_hbm, idx_hbm, val_hbm, out_hbm, idx_v, val_v, flat_v,
          gath_v, out_v, sem):
        wid = lax.axis_index("s") * nc + lax.axis_index("c")
        base_sample = wid * spw

        pltpu.sync_copy(idx_hbm, idx_v)
        pltpu.sync_copy(val_hbm, val_v)

        idx_chunks = []
        for kc in range(_K // _LANES):
            iv = idx_v[pl.ds(kc * _LANES, _LANES)]
            idx_chunks.append(
                lax.shift_left(lax.shift_right_logical(iv, 7), 10)
                + jnp.bitwise_and(iv, 127)
            )

        def build(s, _):
            sg = base_sample + s
            sbase = (
                lax.shift_left(lax.shift_right_logical(sg, 3), 18)
                + lax.shift_left(jnp.bitwise_and(sg, 7), 7)
            )
            for kc in range(_K // _LANES):
                flat_v[pl.ds(s * _K + kc * _LANES, _LANES)] = (
                    sbase + idx_chunks[kc]
                )
            return 0

        lax.fori_loop(0, spw, build, 0)

        copies = []
        for c in range(n_chunks):
            copies.append(
                pltpu.async_copy(
                    table_hbm.at[flat_v.at[pl.ds(c * chunk, chunk)]],
                    gath_v.at[pl.ds(c * chunk, chunk)],
                    sem,
                )
            )
        for cp in copies:
            cp.wait()

        pvals = [
            val_v[pl.ds(kc * _LANES, _LANES)] for kc in range(_K // _LANES)
        ]

        def extract(s, _):
            acc = jnp.zeros((_LANES,), jnp.float32)
            for kc in range(_K // _LANES):
                v = gath_v[pl.ds(s * _K + kc * _LANES, _LANES)]
                acc = acc + v * pvals[kc]
            out_v[s, :] = acc
            return 0

        lax.fori_loop(0, spw, extract, 0)

        pltpu.sync_copy(out_v, out_hbm.at[pl.ds(base_sample, spw)])

    return k(table_flat, idx, vals)


def _finish_body(p_ref, out_ref):
    p = p_ref[...]
    s = jnp.sum(p, axis=1, keepdims=True)
    out_ref[0, 0] = jnp.sum(jnp.log(s)) * (1.0 / _NSAMP)


def _finish(partials):
    out = pl.pallas_call(
        _finish_body,
        out_shape=jax.ShapeDtypeStruct((1, 1), jnp.float32),
        in_specs=[pl.BlockSpec(memory_space=pltpu.VMEM)],
        out_specs=pl.BlockSpec(memory_space=pltpu.SMEM),
    )(partials)
    return out.reshape(())


def kernel(inputs, coef_output_avg, sample_prob_all):
    del inputs
    vals, idx = _topk(coef_output_avg)
    table_flat = (
        sample_prob_all.reshape(_NSAMP // 8, 8, _NCAND // 128, 128)
        .transpose(0, 2, 1, 3)
        .reshape(_NSAMP * _NCAND)
    )
    partials = _sc_gather_call(table_flat, idx, vals)
    return _finish(partials)
